# baseline (speedup 1.0000x reference)
import jax
import jax.numpy as jnp
from jax.experimental import pallas as pl
from jax.experimental.pallas import tpu as pltpu


def _body(in_ref, slow_ref, fast_ref):
    t = pl.program_id(1)
    fast_ref[...] = in_ref[...]
    p = (5 * t + 4) // 21

    @pl.when((21 * p) // 5 == t)
    def _():
        slow_ref[...] = in_ref[...]


def kernel(frames):
    B, T, H, W = frames.shape
    Ts = T // 4

    slow, fast = pl.pallas_call(
        _body,
        grid=(B, T),
        in_specs=[pl.BlockSpec((1, 1, H, W), lambda b, t: (b, t, 0, 0))],
        out_specs=(
            pl.BlockSpec((1, 1, H, W), lambda b, t: (b, (5 * t + 4) // 21, 0, 0)),
            pl.BlockSpec((1, 1, H, W), lambda b, t: (b, t, 0, 0)),
        ),
        out_shape=(
            jax.ShapeDtypeStruct((B, Ts, H, W), frames.dtype),
            jax.ShapeDtypeStruct((B, T, H, W), frames.dtype),
        ),
        compiler_params=pltpu.CompilerParams(
            dimension_semantics=("parallel", "arbitrary"),
        ),
    )(frames)
    return (slow, fast)
